# Initial kernel scaffold; baseline (speedup 1.0000x reference)
import jax
import jax.numpy as jnp
from jax.experimental import pallas as pl


def kernel(user_ids, movie_ids, user_emb, movie_emb, head_w, head_b, user_bias, movie_bias):
    raise NotImplementedError("write your pallas kernel here")


# speedup vs baseline: 3.2263x; 3.2263x over previous
import functools

import jax
import jax.numpy as jnp
from jax import lax
from jax.experimental import pallas as pl
from jax.experimental.pallas import tpu as pltpu
from jax.experimental.pallas import tpu_sc as plsc

B = 16384
EMB = 128
L = 16
NC = 2
NS = 16
NW = NC * NS
W = B // NW
C = 256
NCHUNK = W // C


def _body(uids_hbm, mids_hbm, uemb_hbm, memb_hbm, wu_hbm, wm_hbm, hb_hbm,
          ubias_hbm, mbias_hbm, out_hbm,
          uids_v, mids_v, urows, mrows, wu_v, wm_v, hb_v, ub_v, mb_v, out_v,
          macc, sem_u, sem_m, sem_ub, sem_mb):
  wid = lax.axis_index("s") * NC + lax.axis_index("c")
  base = wid * W

  pltpu.sync_copy(uids_hbm.at[pl.ds(base, W)], uids_v)
  pltpu.sync_copy(mids_hbm.at[pl.ds(base, W)], mids_v)
  pltpu.sync_copy(wu_hbm, wu_v)
  pltpu.sync_copy(wm_hbm, wm_v)
  pltpu.sync_copy(hb_hbm, hb_v)

  cub = pltpu.async_copy(ubias_hbm.at[uids_v], ub_v, sem_ub)
  cmb = pltpu.async_copy(mbias_hbm.at[mids_v], mb_v, sem_mb)

  lanes = lax.iota(jnp.int32, L)
  hb_vec = hb_v[...]

  for g in range(NCHUNK):
    cu = pltpu.async_copy(uemb_hbm.at[uids_v.at[pl.ds(g * C, C)]], urows,
                          sem_u)
    cm = pltpu.async_copy(memb_hbm.at[mids_v.at[pl.ds(g * C, C)]], mrows,
                          sem_m)
    cu.wait()
    cm.wait()
    if g == 0:
      cub.wait()
      cmb.wait()

    def group(j, _):
      gbase = j * L
      ib = g * C + gbase
      for t in range(L):
        acc = jnp.zeros((L,), jnp.float32)
        for k in range(EMB // L):
          sl = pl.ds(k * L, L)
          acc = acc + urows[gbase + t, sl] * wu_v[sl]
          acc = acc + mrows[gbase + t, sl] * wm_v[sl]
        plsc.store_scatter(macc, [lanes, jnp.full((L,), t, jnp.int32)], acc)
      out16 = ub_v[pl.ds(ib, L)] + mb_v[pl.ds(ib, L)] + hb_vec
      for t in range(L):
        out16 = out16 + macc[t]
      out_v[pl.ds(ib, L)] = out16
      return 0

    lax.fori_loop(0, C // L, group, 0)

  pltpu.sync_copy(out_v, out_hbm.at[pl.ds(base, W)])


_mesh = plsc.VectorSubcoreMesh(core_axis_name="c", subcore_axis_name="s")

_sc_call = functools.partial(
    pl.kernel,
    out_type=jax.ShapeDtypeStruct((B,), jnp.float32),
    mesh=_mesh,
    compiler_params=pltpu.CompilerParams(needs_layout_passes=False),
    scratch_types=[
        pltpu.VMEM((W,), jnp.int32),
        pltpu.VMEM((W,), jnp.int32),
        pltpu.VMEM((C, EMB), jnp.float32),
        pltpu.VMEM((C, EMB), jnp.float32),
        pltpu.VMEM((EMB,), jnp.float32),
        pltpu.VMEM((EMB,), jnp.float32),
        pltpu.VMEM((L,), jnp.float32),
        pltpu.VMEM((W,), jnp.float32),
        pltpu.VMEM((W,), jnp.float32),
        pltpu.VMEM((W,), jnp.float32),
        pltpu.VMEM((L, L), jnp.float32),
        pltpu.SemaphoreType.DMA,
        pltpu.SemaphoreType.DMA,
        pltpu.SemaphoreType.DMA,
        pltpu.SemaphoreType.DMA,
    ],
)(_body)


@jax.jit
def kernel(user_ids, movie_ids, user_emb, movie_emb, head_w, head_b,
           user_bias, movie_bias):
  uids = user_ids.astype(jnp.int32)
  mids = movie_ids.astype(jnp.int32)
  wu = head_w[0, :EMB]
  wm = head_w[0, EMB:]
  hb = jnp.broadcast_to(head_b, (L,))
  return _sc_call(uids, mids, user_emb, movie_emb, wu, wm, hb,
                  user_bias.reshape(-1), movie_bias.reshape(-1))


# speedup vs baseline: 3.4226x; 1.0608x over previous
import functools

import jax
import jax.numpy as jnp
from jax import lax
from jax.experimental import pallas as pl
from jax.experimental.pallas import tpu as pltpu
from jax.experimental.pallas import tpu_sc as plsc

B = 16384
EMB = 128
L = 16
NC = 2
NS = 16
NW = NC * NS
W = B // NW
C = 128
NCHUNK = W // C
NK = EMB // L


def _body(uids_hbm, mids_hbm, uemb_hbm, memb_hbm, wu_hbm, wm_hbm, hb_hbm,
          ubias_hbm, mbias_hbm, out_hbm,
          uids_v, mids_v, u0, u1, m0, m1, wu_v, wm_v, hb_v, ub_v, mb_v, out_v,
          macc, sem_u0, sem_u1, sem_m0, sem_m1, sem_ub, sem_mb):
  wid = lax.axis_index("s") * NC + lax.axis_index("c")
  base = wid * W

  pltpu.sync_copy(uids_hbm.at[pl.ds(base, W)], uids_v)
  pltpu.sync_copy(mids_hbm.at[pl.ds(base, W)], mids_v)
  pltpu.sync_copy(wu_hbm, wu_v)
  pltpu.sync_copy(wm_hbm, wm_v)
  pltpu.sync_copy(hb_hbm, hb_v)

  cub = pltpu.async_copy(ubias_hbm.at[uids_v], ub_v, sem_ub)
  cmb = pltpu.async_copy(mbias_hbm.at[mids_v], mb_v, sem_mb)

  lanes = lax.iota(jnp.int32, L)
  hb_vec = hb_v[...]
  wu_r = [wu_v[pl.ds(k * L, L)] for k in range(NK)]
  wm_r = [wm_v[pl.ds(k * L, L)] for k in range(NK)]

  ubufs = [u0, u1]
  mbufs = [m0, m1]
  usems = [sem_u0, sem_u1]
  msems = [sem_m0, sem_m1]

  def issue(g):
    b = g % 2
    cu = pltpu.async_copy(uemb_hbm.at[uids_v.at[pl.ds(g * C, C)]], ubufs[b],
                          usems[b])
    cm = pltpu.async_copy(memb_hbm.at[mids_v.at[pl.ds(g * C, C)]], mbufs[b],
                          msems[b])
    return cu, cm

  inflight = issue(0)

  for g in range(NCHUNK):
    cu, cm = inflight
    if g + 1 < NCHUNK:
      inflight = issue(g + 1)
    cu.wait()
    cm.wait()
    if g == 0:
      cub.wait()
      cmb.wait()
    urows = ubufs[g % 2]
    mrows = mbufs[g % 2]

    def group(j, _):
      gbase = j * L
      ib = g * C + gbase
      for t in range(L):
        acc = urows[gbase + t, pl.ds(0, L)] * wu_r[0]
        for k in range(1, NK):
          acc = acc + urows[gbase + t, pl.ds(k * L, L)] * wu_r[k]
        for k in range(NK):
          acc = acc + mrows[gbase + t, pl.ds(k * L, L)] * wm_r[k]
        plsc.store_scatter(macc, [lanes, jnp.full((L,), t, jnp.int32)], acc)
      out16 = ub_v[pl.ds(ib, L)] + mb_v[pl.ds(ib, L)] + hb_vec
      for t in range(L):
        out16 = out16 + macc[t]
      out_v[pl.ds(ib, L)] = out16
      return 0

    lax.fori_loop(0, C // L, group, 0)

  pltpu.sync_copy(out_v, out_hbm.at[pl.ds(base, W)])


_mesh = plsc.VectorSubcoreMesh(core_axis_name="c", subcore_axis_name="s")

_sc_call = functools.partial(
    pl.kernel,
    out_type=jax.ShapeDtypeStruct((B,), jnp.float32),
    mesh=_mesh,
    compiler_params=pltpu.CompilerParams(needs_layout_passes=False),
    scratch_types=[
        pltpu.VMEM((W,), jnp.int32),
        pltpu.VMEM((W,), jnp.int32),
        pltpu.VMEM((C, EMB), jnp.float32),
        pltpu.VMEM((C, EMB), jnp.float32),
        pltpu.VMEM((C, EMB), jnp.float32),
        pltpu.VMEM((C, EMB), jnp.float32),
        pltpu.VMEM((EMB,), jnp.float32),
        pltpu.VMEM((EMB,), jnp.float32),
        pltpu.VMEM((L,), jnp.float32),
        pltpu.VMEM((W,), jnp.float32),
        pltpu.VMEM((W,), jnp.float32),
        pltpu.VMEM((W,), jnp.float32),
        pltpu.VMEM((L, L), jnp.float32),
        pltpu.SemaphoreType.DMA,
        pltpu.SemaphoreType.DMA,
        pltpu.SemaphoreType.DMA,
        pltpu.SemaphoreType.DMA,
        pltpu.SemaphoreType.DMA,
        pltpu.SemaphoreType.DMA,
    ],
)(_body)


@jax.jit
def kernel(user_ids, movie_ids, user_emb, movie_emb, head_w, head_b,
           user_bias, movie_bias):
  uids = user_ids.astype(jnp.int32)
  mids = movie_ids.astype(jnp.int32)
  wu = head_w[0, :EMB]
  wm = head_w[0, EMB:]
  hb = jnp.broadcast_to(head_b, (L,))
  return _sc_call(uids, mids, user_emb, movie_emb, wu, wm, hb,
                  user_bias.reshape(-1), movie_bias.reshape(-1))


# speedup vs baseline: 3.6309x; 1.0609x over previous
import functools

import jax
import jax.numpy as jnp
from jax import lax
from jax.experimental import pallas as pl
from jax.experimental.pallas import tpu as pltpu
from jax.experimental.pallas import tpu_sc as plsc

B = 16384
EMB = 128
L = 16
NC = 2
NS = 16
NW = NC * NS
W = B // NW
C = 128
NCHUNK = W // C
NK = EMB // L


def _body(uids_hbm, mids_hbm, uemb_hbm, memb_hbm, wu_hbm, wm_hbm, hb_hbm,
          ubias_hbm, mbias_hbm, out_hbm,
          uids_v, mids_v, u0, u1, m0, m1, wu_v, wm_v, hb_v, ub_v, mb_v, out_v,
          macc, sem_u0, sem_u1, sem_m0, sem_m1, sem_ub, sem_mb):
  wid = lax.axis_index("s") * NC + lax.axis_index("c")
  base = wid * W

  pltpu.sync_copy(uids_hbm.at[pl.ds(base, W)], uids_v)
  pltpu.sync_copy(mids_hbm.at[pl.ds(base, W)], mids_v)
  pltpu.sync_copy(wu_hbm, wu_v)
  pltpu.sync_copy(wm_hbm, wm_v)
  pltpu.sync_copy(hb_hbm, hb_v)

  cub = pltpu.async_copy(ubias_hbm.at[uids_v], ub_v, sem_ub)
  cmb = pltpu.async_copy(mbias_hbm.at[mids_v], mb_v, sem_mb)

  lanes = lax.iota(jnp.int32, L)
  hb_vec = hb_v[...]
  wu_r = [wu_v[pl.ds(k * L, L)] for k in range(NK)]
  wm_r = [wm_v[pl.ds(k * L, L)] for k in range(NK)]

  ubufs = [u0, u1]
  mbufs = [m0, m1]
  usems = [sem_u0, sem_u1]
  msems = [sem_m0, sem_m1]

  def issue(g):
    b = g % 2
    cu = pltpu.async_copy(uemb_hbm.at[uids_v.at[pl.ds(g * C, C)]], ubufs[b],
                          usems[b])
    cm = pltpu.async_copy(memb_hbm.at[mids_v.at[pl.ds(g * C, C)]], mbufs[b],
                          msems[b])
    return cu, cm

  inflight = issue(0)

  for g in range(NCHUNK):
    cu, cm = inflight
    if g + 1 < NCHUNK:
      inflight = issue(g + 1)
    cu.wait()
    cm.wait()
    if g == 0:
      cub.wait()
      cmb.wait()
    urows = ubufs[g % 2]
    mrows = mbufs[g % 2]

    def group(j, _):
      gbase = j * L
      ib = g * C + gbase
      for t in range(L):
        prods = [urows[gbase + t, pl.ds(k * L, L)] * wu_r[k]
                 for k in range(NK)]
        prods += [mrows[gbase + t, pl.ds(k * L, L)] * wm_r[k]
                  for k in range(NK)]
        while len(prods) > 1:
          prods = [prods[i] + prods[i + 1] for i in range(0, len(prods), 2)]
        plsc.store_scatter(macc, [lanes, jnp.full((L,), t, jnp.int32)],
                           prods[0])
      out16 = ub_v[pl.ds(ib, L)] + mb_v[pl.ds(ib, L)] + hb_vec
      for t in range(L):
        out16 = out16 + macc[t]
      out_v[pl.ds(ib, L)] = out16
      return 0

    lax.fori_loop(0, C // L, group, 0)

  pltpu.sync_copy(out_v, out_hbm.at[pl.ds(base, W)])


_mesh = plsc.VectorSubcoreMesh(core_axis_name="c", subcore_axis_name="s")

_sc_call = functools.partial(
    pl.kernel,
    out_type=jax.ShapeDtypeStruct((B,), jnp.float32),
    mesh=_mesh,
    compiler_params=pltpu.CompilerParams(needs_layout_passes=False),
    scratch_types=[
        pltpu.VMEM((W,), jnp.int32),
        pltpu.VMEM((W,), jnp.int32),
        pltpu.VMEM((C, EMB), jnp.float32),
        pltpu.VMEM((C, EMB), jnp.float32),
        pltpu.VMEM((C, EMB), jnp.float32),
        pltpu.VMEM((C, EMB), jnp.float32),
        pltpu.VMEM((EMB,), jnp.float32),
        pltpu.VMEM((EMB,), jnp.float32),
        pltpu.VMEM((L,), jnp.float32),
        pltpu.VMEM((W,), jnp.float32),
        pltpu.VMEM((W,), jnp.float32),
        pltpu.VMEM((W,), jnp.float32),
        pltpu.VMEM((L, L), jnp.float32),
        pltpu.SemaphoreType.DMA,
        pltpu.SemaphoreType.DMA,
        pltpu.SemaphoreType.DMA,
        pltpu.SemaphoreType.DMA,
        pltpu.SemaphoreType.DMA,
        pltpu.SemaphoreType.DMA,
    ],
)(_body)


@jax.jit
def kernel(user_ids, movie_ids, user_emb, movie_emb, head_w, head_b,
           user_bias, movie_bias):
  uids = user_ids.astype(jnp.int32)
  mids = movie_ids.astype(jnp.int32)
  wu = head_w[0, :EMB]
  wm = head_w[0, EMB:]
  hb = jnp.broadcast_to(head_b, (L,))
  return _sc_call(uids, mids, user_emb, movie_emb, wu, wm, hb,
                  user_bias.reshape(-1), movie_bias.reshape(-1))
